# baseline (speedup 1.0000x reference)
import functools

import jax
import jax.numpy as jnp
from jax import lax
from jax.experimental import pallas as pl
from jax.experimental.pallas import tpu as pltpu
from jax.experimental.pallas import tpu_sc as plsc

VOCAB = 1000
EMB_DIM = 64
BATCH = 4096
HIST = 50

BPC = 2
CHUNK = BPC * HIST
CPAD = 104
NCHUNKS = BATCH // BPC

NBUF = 8
LAG = 4


def _make_kernel():
    info = plsc.get_sparse_core_info()
    nc, ns = info.num_cores, info.num_subcores
    nw = nc * ns
    cpw = NCHUNKS // nw
    nouter = cpw // NBUF

    mesh = plsc.VectorSubcoreMesh(core_axis_name="c", subcore_axis_name="s")

    scratch = (
        [pltpu.VMEM((cpw, CPAD), jnp.int32)]
        + [pltpu.VMEM((CPAD, EMB_DIM), jnp.float32) for _ in range(NBUF)]
        + [pltpu.SemaphoreType.DMA for _ in range(2 * NBUF)]
    )

    @functools.partial(
        pl.kernel,
        mesh=mesh,
        out_type=jax.ShapeDtypeStruct((BATCH, HIST, EMB_DIM), jnp.float32),
        scratch_types=scratch,
        compiler_params=pltpu.CompilerParams(use_tc_tiling_on_sc=False),
    )
    def emb_kernel(idx_hbm, table_hbm, out_hbm, idx_v, *bufs_and_sems):
        bufs = bufs_and_sems[:NBUF]
        sem_g = bufs_and_sems[NBUF:2 * NBUF]
        sem_o = bufs_and_sems[2 * NBUF:]

        wid = lax.axis_index("s") * nc + lax.axis_index("c")
        base = wid * cpw
        pltpu.sync_copy(idx_hbm.at[wid], idx_v)

        def gather(chunk, b):
            pltpu.async_copy(table_hbm.at[idx_v.at[chunk]], bufs[b], sem_g[b])

        def put(chunk, b):
            row0 = (base + chunk) * BPC
            for r in range(BPC):
                pltpu.async_copy(
                    bufs[b].at[pl.ds(r * HIST, HIST)],
                    out_hbm.at[row0 + r], sem_o[b])

        def wait_gather(b):
            pltpu.make_async_copy(
                table_hbm.at[idx_v.at[0]], bufs[b], sem_g[b]).wait()

        def wait_put(b):
            for r in range(BPC):
                pltpu.make_async_copy(
                    bufs[b].at[pl.ds(r * HIST, HIST)],
                    out_hbm.at[r], sem_o[b]).wait()

        for b in range(LAG):
            gather(b, b)

        def round_body(i, carry):
            for b in range(NBUF):
                j = i * NBUF + b
                k = j + LAG
                bk = (b + LAG) % NBUF

                @pl.when(k < cpw)
                def _():
                    @pl.when(k >= NBUF)
                    def _():
                        wait_put(bk)

                    gather(k, bk)

                wait_gather(b)
                put(j, b)
            return carry

        lax.fori_loop(0, nouter, round_body, 0)

        for b in range(NBUF):
            wait_put(b)

    return emb_kernel, nw


_emb_kernel, _NW = _make_kernel()


def kernel(indices, table):
    idx = indices.reshape(_NW, NCHUNKS // _NW, CHUNK)
    idx = jnp.pad(idx, ((0, 0), (0, 0), (0, CPAD - CHUNK)))
    return _emb_kernel(idx, table)


# speedup vs baseline: 1.1358x; 1.1358x over previous
import functools

import jax
import jax.numpy as jnp
from jax import lax
from jax.experimental import pallas as pl
from jax.experimental.pallas import tpu as pltpu
from jax.experimental.pallas import tpu_sc as plsc

VOCAB = 1000
EMB_DIM = 64
BATCH = 4096
HIST = 50

BBLK = 128
NBT = BATCH // BBLK
NBLOCKS = HIST * NBT
ETILES = EMB_DIM // 8


def _make_kernel():
    info = plsc.get_sparse_core_info()
    nc, ns, nl = info.num_cores, info.num_subcores, info.num_lanes
    nw = nc * ns
    bpw = NBLOCKS // nw

    mesh = plsc.VectorSubcoreMesh(core_axis_name="c", subcore_axis_name="s")

    scratch = [
        pltpu.VMEM((VOCAB * EMB_DIM,), jnp.float32),
        pltpu.VMEM((bpw, BBLK), jnp.int32),
        pltpu.VMEM((ETILES, 8, BBLK), jnp.float32),
        pltpu.SemaphoreType.DMA,
        pltpu.SemaphoreType.DMA,
    ]

    @functools.partial(
        pl.kernel,
        mesh=mesh,
        out_type=jax.ShapeDtypeStruct(
            (HIST, ETILES, NBT, 8, BBLK), jnp.float32),
        scratch_types=scratch,
        compiler_params=pltpu.CompilerParams(
            use_tc_tiling_on_sc=False, needs_layout_passes=False),
    )
    def emb_kernel(idx_hbm, table_hbm, out_hbm, table_v, idx_v, buf, sem_t,
                   sem_o):
        wid = lax.axis_index("s") * nc + lax.axis_index("c")
        base = wid * bpw
        pltpu.async_copy(table_hbm, table_v, sem_t)
        pltpu.sync_copy(idx_hbm.at[wid], idx_v)
        pltpu.make_async_copy(table_hbm, table_v, sem_t).wait()

        def block_body(blk, carry):
            flat = base + blk
            h = flat // NBT
            bt = lax.rem(flat, NBT)

            for j in range(BBLK // nl):
                idx64 = idx_v[blk, pl.ds(nl * j, nl)] * EMB_DIM
                for et in range(ETILES):
                    for ei in range(8):
                        v = plsc.load_gather(
                            table_v, [idx64 + (et * 8 + ei)])
                        buf[et, ei, pl.ds(nl * j, nl)] = v

            for et in range(ETILES):
                pltpu.async_copy(
                    buf.at[et], out_hbm.at[h, et, bt], sem_o)
            for et in range(ETILES):
                pltpu.make_async_copy(
                    buf.at[et], out_hbm.at[0, 0, 0], sem_o).wait()
            return carry

        lax.fori_loop(0, bpw, block_body, 0)

    return emb_kernel, nw


_emb_kernel, _NW = _make_kernel()


def kernel(indices, table):
    idxt = indices.T.reshape(_NW, NBLOCKS // _NW, BBLK)
    out5 = _emb_kernel(idxt, table.reshape(VOCAB * EMB_DIM))
    return out5.transpose(2, 4, 0, 1, 3).reshape(BATCH, HIST, EMB_DIM)


# speedup vs baseline: 1.4678x; 1.2923x over previous
import functools

import jax
import jax.numpy as jnp
from jax import lax
from jax.experimental import pallas as pl
from jax.experimental.pallas import tpu as pltpu
from jax.experimental.pallas import tpu_sc as plsc

VOCAB = 1000
EMB_DIM = 64
BATCH = 4096
HIST = 50

BBLK = 128
NBT = BATCH // BBLK
NBLOCKS = HIST * NBT
ETILES = EMB_DIM // 8


def _make_kernel():
    info = plsc.get_sparse_core_info()
    nc, ns, nl = info.num_cores, info.num_subcores, info.num_lanes
    nw = nc * ns
    bpw = NBLOCKS // nw

    mesh = plsc.VectorSubcoreMesh(core_axis_name="c", subcore_axis_name="s")

    scratch = [
        pltpu.VMEM((VOCAB * EMB_DIM,), jnp.float32),
        pltpu.VMEM((bpw, BBLK), jnp.int32),
        pltpu.VMEM((2, ETILES, 8, BBLK), jnp.float32),
        pltpu.SemaphoreType.DMA,
        pltpu.SemaphoreType.DMA,
    ]

    @functools.partial(
        pl.kernel,
        mesh=mesh,
        out_type=jax.ShapeDtypeStruct(
            (HIST, ETILES, NBT, 8, BBLK), jnp.float32),
        scratch_types=scratch,
        compiler_params=pltpu.CompilerParams(
            use_tc_tiling_on_sc=False, needs_layout_passes=False),
    )
    def emb_kernel(idx_hbm, table_hbm, out_hbm, table_v, idx_v, buf, sem_t,
                   sem_o):
        wid = lax.axis_index("s") * nc + lax.axis_index("c")
        base = wid * bpw
        pltpu.async_copy(table_hbm, table_v, sem_t)
        pltpu.sync_copy(idx_hbm.at[wid], idx_v)
        pltpu.make_async_copy(table_hbm, table_v, sem_t).wait()

        ngrp = BBLK // nl

        def block_body(blk, carry):
            p = lax.rem(blk, 2)
            flat = base + blk
            h = flat // NBT
            bt = lax.rem(flat, NBT)

            @pl.when(blk >= 2)
            def _():
                for et in range(ETILES):
                    pltpu.make_async_copy(
                        buf.at[0, et], out_hbm.at[0, 0, 0], sem_o).wait()

            idxb = [
                idx_v[blk, pl.ds(nl * j, nl)] * EMB_DIM for j in range(ngrp)
            ]
            for e in range(EMB_DIM):
                vs = [
                    plsc.load_gather(table_v, [idxb[j] + e])
                    for j in range(ngrp)
                ]
                for j in range(ngrp):
                    buf[p, e // 8, e % 8, pl.ds(nl * j, nl)] = vs[j]

            for et in range(ETILES):
                pltpu.async_copy(
                    buf.at[p, et], out_hbm.at[h, et, bt], sem_o)
            return carry

        lax.fori_loop(0, bpw, block_body, 0)

        for et in range(2 * ETILES):
            pltpu.make_async_copy(
                buf.at[0, 0], out_hbm.at[0, 0, 0], sem_o).wait()

    return emb_kernel, nw


_emb_kernel, _NW = _make_kernel()


def kernel(indices, table):
    idxt = indices.T.reshape(_NW, NBLOCKS // _NW, BBLK)
    out5 = _emb_kernel(idxt, table.reshape(VOCAB * EMB_DIM))
    return out5.transpose(2, 4, 0, 1, 3).reshape(BATCH, HIST, EMB_DIM)


# speedup vs baseline: 6.3969x; 4.3581x over previous
import functools

import jax
import jax.numpy as jnp
from jax import lax
from jax.experimental import pallas as pl
from jax.experimental.pallas import tpu as pltpu
from jax.experimental.pallas import tpu_sc as plsc

VOCAB = 1000
EMB_DIM = 64
BATCH = 4096
HIST = 50

BBLK = 128
NBT = BATCH // BBLK
NBLOCKS = HIST * NBT
ETILES = EMB_DIM // 8
VPAD = 1024


def _make_kernel():
    info = plsc.get_sparse_core_info()
    nc, ns, nl = info.num_cores, info.num_subcores, info.num_lanes
    nw = nc * ns
    bpw = NBLOCKS // nw

    mesh = plsc.VectorSubcoreMesh(core_axis_name="c", subcore_axis_name="s")

    scratch = [
        pltpu.VMEM((EMB_DIM * VPAD,), jnp.float32),
        pltpu.VMEM((bpw, BBLK), jnp.int32),
        pltpu.VMEM((2, ETILES, 8, BBLK), jnp.float32),
        pltpu.SemaphoreType.DMA,
        pltpu.SemaphoreType.DMA,
    ]

    @functools.partial(
        pl.kernel,
        mesh=mesh,
        out_type=jax.ShapeDtypeStruct(
            (HIST, ETILES, NBT, 8, BBLK), jnp.float32),
        scratch_types=scratch,
        compiler_params=pltpu.CompilerParams(
            use_tc_tiling_on_sc=False, needs_layout_passes=False),
    )
    def emb_kernel(idx_hbm, table_hbm, out_hbm, table_v, idx_v, buf, sem_t,
                   sem_o):
        wid = lax.axis_index("s") * nc + lax.axis_index("c")
        base = wid * bpw
        pltpu.async_copy(table_hbm, table_v, sem_t)
        pltpu.sync_copy(idx_hbm.at[wid], idx_v)
        pltpu.make_async_copy(table_hbm, table_v, sem_t).wait()

        ngrp = BBLK // nl

        def block_body(blk, carry):
            p = lax.rem(blk, 2)
            flat = base + blk
            h = flat // NBT
            bt = lax.rem(flat, NBT)

            @pl.when(blk >= 2)
            def _():
                for et in range(ETILES):
                    pltpu.make_async_copy(
                        buf.at[0, et], out_hbm.at[0, 0, 0], sem_o).wait()

            idxb = [idx_v[blk, pl.ds(nl * j, nl)] for j in range(ngrp)]
            for e in range(EMB_DIM):
                vs = [
                    plsc.load_gather(table_v, [idxb[j] + (e * VPAD)])
                    for j in range(ngrp)
                ]
                for j in range(ngrp):
                    buf[p, e // 8, e % 8, pl.ds(nl * j, nl)] = vs[j]

            for et in range(ETILES):
                pltpu.async_copy(
                    buf.at[p, et], out_hbm.at[h, et, bt], sem_o)
            return carry

        lax.fori_loop(0, bpw, block_body, 0)

        for et in range(2 * ETILES):
            pltpu.make_async_copy(
                buf.at[0, 0], out_hbm.at[0, 0, 0], sem_o).wait()

    return emb_kernel, nw


_emb_kernel, _NW = _make_kernel()


def kernel(indices, table):
    idxt = indices.T.reshape(_NW, NBLOCKS // _NW, BBLK)
    tpad = jnp.pad(table.T, ((0, 0), (0, VPAD - VOCAB))).reshape(
        EMB_DIM * VPAD)
    out5 = _emb_kernel(idxt, tpad)
    return out5.transpose(2, 4, 0, 1, 3).reshape(BATCH, HIST, EMB_DIM)


# speedup vs baseline: 7.2268x; 1.1297x over previous
import functools

import jax
import jax.numpy as jnp
from jax import lax
from jax.experimental import pallas as pl
from jax.experimental.pallas import tpu as pltpu
from jax.experimental.pallas import tpu_sc as plsc

VOCAB = 1000
EMB_DIM = 64
BATCH = 4096
HIST = 50

BBLK = 128
NBT = BATCH // BBLK
NBLOCKS = HIST * NBT
ETILES = EMB_DIM // 8
VPAD = 1024


def _make_kernel():
    info = plsc.get_sparse_core_info()
    nc, ns, nl = info.num_cores, info.num_subcores, info.num_lanes
    nw = nc * ns
    bpw = NBLOCKS // nw

    mesh = plsc.VectorSubcoreMesh(core_axis_name="c", subcore_axis_name="s")

    scratch = [
        pltpu.VMEM((EMB_DIM * VPAD,), jnp.float32),
        pltpu.VMEM((bpw, BBLK), jnp.int32),
        pltpu.VMEM((2, ETILES, 8, BBLK), jnp.float32),
        pltpu.SemaphoreType.DMA,
        pltpu.SemaphoreType.DMA,
    ]

    @functools.partial(
        pl.kernel,
        mesh=mesh,
        out_type=jax.ShapeDtypeStruct(
            (HIST, ETILES, NBT, 8, BBLK), jnp.float32),
        scratch_types=scratch,
        compiler_params=pltpu.CompilerParams(
            use_tc_tiling_on_sc=False, needs_layout_passes=False),
    )
    def emb_kernel(idx_hbm, table_hbm, out_hbm, table_v, idx_v, buf, sem_t,
                   sem_o):
        wid = lax.axis_index("s") * nc + lax.axis_index("c")
        base = wid * bpw
        pltpu.async_copy(table_hbm, table_v, sem_t)
        pltpu.sync_copy(idx_hbm.at[wid], idx_v)
        pltpu.make_async_copy(table_hbm, table_v, sem_t).wait()

        ngrp = BBLK // nl

        def block_body(blk, carry):
            p = lax.rem(blk, 2)
            flat = base + blk
            h = flat // NBT
            bt = lax.rem(flat, NBT)

            @pl.when(blk >= 2)
            def _():
                for et in range(ETILES):
                    pltpu.make_async_copy(
                        buf.at[0, et], out_hbm.at[0, 0, 0], sem_o).wait()

            idxb = [idx_v[blk, pl.ds(nl * j, nl)] for j in range(ngrp)]

            def gathers(e):
                return [
                    plsc.load_gather(table_v, [idxb[j] + (e * VPAD)])
                    for j in range(ngrp)
                ]

            def stores(e, vs):
                for j in range(ngrp):
                    buf[p, e // 8, e % 8, pl.ds(nl * j, nl)] = vs[j]

            vs0 = gathers(0)
            vs1 = gathers(1)
            for e in range(2, EMB_DIM):
                vs2 = []
                for j in range(ngrp):
                    vs2.append(
                        plsc.load_gather(table_v, [idxb[j] + (e * VPAD)]))
                    buf[p, (e - 2) // 8, (e - 2) % 8,
                        pl.ds(nl * j, nl)] = vs0[j]
                vs0, vs1 = vs1, vs2
            stores(EMB_DIM - 2, vs0)
            stores(EMB_DIM - 1, vs1)

            for et in range(ETILES):
                pltpu.async_copy(
                    buf.at[p, et], out_hbm.at[h, et, bt], sem_o)
            return carry

        lax.fori_loop(0, bpw, block_body, 0)

        for et in range(2 * ETILES):
            pltpu.make_async_copy(
                buf.at[0, 0], out_hbm.at[0, 0, 0], sem_o).wait()

    return emb_kernel, nw


_emb_kernel, _NW = _make_kernel()


def kernel(indices, table):
    idxt = indices.T.reshape(_NW, NBLOCKS // _NW, BBLK)
    tpad = jnp.pad(table.T, ((0, 0), (0, VPAD - VOCAB))).reshape(
        EMB_DIM * VPAD)
    out5 = _emb_kernel(idxt, tpad)
    return out5.transpose(2, 4, 0, 1, 3).reshape(BATCH, HIST, EMB_DIM)


# speedup vs baseline: 7.3198x; 1.0129x over previous
import functools

import jax
import jax.numpy as jnp
from jax import lax
from jax.experimental import pallas as pl
from jax.experimental.pallas import tpu as pltpu
from jax.experimental.pallas import tpu_sc as plsc

VOCAB = 1000
EMB_DIM = 64
BATCH = 4096
HIST = 50

BBLK = 128
NBT = BATCH // BBLK
NBLOCKS = HIST * NBT
ETILES = EMB_DIM // 8
VPAD = 1024


def _make_kernel():
    info = plsc.get_sparse_core_info()
    nc, ns, nl = info.num_cores, info.num_subcores, info.num_lanes
    nw = nc * ns
    bpw = NBLOCKS // nw

    mesh = plsc.VectorSubcoreMesh(core_axis_name="c", subcore_axis_name="s")

    scratch = [
        pltpu.VMEM((EMB_DIM * VPAD,), jnp.float32),
        pltpu.VMEM((bpw, BBLK), jnp.int32),
        pltpu.VMEM((4, ETILES, 8, BBLK), jnp.float32),
        pltpu.SemaphoreType.DMA,
        pltpu.SemaphoreType.DMA,
    ]

    @functools.partial(
        pl.kernel,
        mesh=mesh,
        out_type=jax.ShapeDtypeStruct(
            (HIST, ETILES, NBT, 8, BBLK), jnp.float32),
        scratch_types=scratch,
        compiler_params=pltpu.CompilerParams(
            use_tc_tiling_on_sc=False, needs_layout_passes=False),
    )
    def emb_kernel(idx_hbm, table_hbm, out_hbm, table_v, idx_v, buf, sem_t,
                   sem_o):
        wid = lax.axis_index("s") * nc + lax.axis_index("c")
        base = wid * bpw
        pltpu.async_copy(table_hbm, table_v, sem_t)
        pltpu.sync_copy(idx_hbm.at[wid], idx_v)
        pltpu.make_async_copy(table_hbm, table_v, sem_t).wait()

        ngrp = BBLK // nl

        def block_body(blk, carry):
            p = lax.rem(blk, 4)
            flat = base + blk
            h = flat // NBT
            bt = lax.rem(flat, NBT)

            @pl.when(blk >= 4)
            def _():
                pltpu.make_async_copy(
                    buf.at[0], out_hbm.at[0, pl.ds(0, ETILES), 0],
                    sem_o).wait()

            idxb = [idx_v[blk, pl.ds(nl * j, nl)] for j in range(ngrp)]

            def gathers(e):
                return [
                    plsc.load_gather(table_v, [idxb[j] + (e * VPAD)])
                    for j in range(ngrp)
                ]

            def stores(e, vs):
                for j in range(ngrp):
                    buf[p, e // 8, e % 8, pl.ds(nl * j, nl)] = vs[j]

            vs0 = gathers(0)
            vs1 = gathers(1)
            for e in range(2, EMB_DIM):
                vs2 = []
                for j in range(ngrp):
                    vs2.append(
                        plsc.load_gather(table_v, [idxb[j] + (e * VPAD)]))
                    buf[p, (e - 2) // 8, (e - 2) % 8,
                        pl.ds(nl * j, nl)] = vs0[j]
                vs0, vs1 = vs1, vs2
            stores(EMB_DIM - 2, vs0)
            stores(EMB_DIM - 1, vs1)

            pltpu.async_copy(
                buf.at[p], out_hbm.at[h, pl.ds(0, ETILES), bt], sem_o)
            return carry

        lax.fori_loop(0, bpw, block_body, 0)

        for _ in range(4):
            pltpu.make_async_copy(
                buf.at[0], out_hbm.at[0, pl.ds(0, ETILES), 0],
                sem_o).wait()

    return emb_kernel, nw


_emb_kernel, _NW = _make_kernel()


def kernel(indices, table):
    idxt = indices.T.reshape(_NW, NBLOCKS // _NW, BBLK)
    tpad = jnp.pad(table.T, ((0, 0), (0, VPAD - VOCAB))).reshape(
        EMB_DIM * VPAD)
    out5 = _emb_kernel(idxt, tpad)
    return out5.transpose(2, 4, 0, 1, 3).reshape(BATCH, HIST, EMB_DIM)


# speedup vs baseline: 8.1257x; 1.1101x over previous
import functools

import jax
import jax.numpy as jnp
from jax import lax
from jax.experimental import pallas as pl
from jax.experimental.pallas import tpu as pltpu
from jax.experimental.pallas import tpu_sc as plsc

VOCAB = 1000
EMB_DIM = 64
BATCH = 4096
HIST = 50

BBLK = 128
NBT = BATCH // BBLK
NBLOCKS = HIST * NBT
ETILES = EMB_DIM // 8
TSTRIDE = VOCAB


def _make_kernel():
    info = plsc.get_sparse_core_info()
    nc, ns, nl = info.num_cores, info.num_subcores, info.num_lanes
    nw = nc * ns
    bpw = NBLOCKS // nw

    mesh = plsc.VectorSubcoreMesh(core_axis_name="c", subcore_axis_name="s")

    scratch = [
        pltpu.VMEM((EMB_DIM * VOCAB,), jnp.float32),
        pltpu.VMEM((bpw, BBLK), jnp.int32),
        pltpu.VMEM((4, ETILES, 8, BBLK), jnp.float32),
        pltpu.SemaphoreType.DMA,
        pltpu.SemaphoreType.DMA,
    ]

    @functools.partial(
        pl.kernel,
        mesh=mesh,
        out_type=jax.ShapeDtypeStruct(
            (HIST, ETILES, NBT, 8, BBLK), jnp.float32),
        scratch_types=scratch,
        compiler_params=pltpu.CompilerParams(
            use_tc_tiling_on_sc=False, needs_layout_passes=False),
    )
    def emb_kernel(idx_hbm, table_hbm, out_hbm, table_v, idx_v, buf, sem_t,
                   sem_o):
        wid = lax.axis_index("s") * nc + lax.axis_index("c")
        base = wid * bpw
        pltpu.async_copy(table_hbm, table_v, sem_t)
        pltpu.sync_copy(idx_hbm.at[wid], idx_v)
        pltpu.make_async_copy(table_hbm, table_v, sem_t).wait()

        ngrp = BBLK // nl

        def block_body(blk, carry):
            p = lax.rem(blk, 4)
            flat = base + blk
            h = flat // NBT
            bt = lax.rem(flat, NBT)

            @pl.when(blk >= 4)
            def _():
                pltpu.make_async_copy(
                    buf.at[0], out_hbm.at[0, pl.ds(0, ETILES), 0],
                    sem_o).wait()

            idxb = [idx_v[blk, pl.ds(nl * j, nl)] for j in range(ngrp)]

            def gathers(e):
                return [
                    plsc.load_gather(table_v, [idxb[j] + (e * TSTRIDE)])
                    for j in range(ngrp)
                ]

            def stores(e, vs):
                for j in range(ngrp):
                    buf[p, e // 8, e % 8, pl.ds(nl * j, nl)] = vs[j]

            vs0 = gathers(0)
            vs1 = gathers(1)
            for e in range(2, EMB_DIM):
                vs2 = []
                for j in range(ngrp):
                    vs2.append(
                        plsc.load_gather(table_v, [idxb[j] + (e * TSTRIDE)]))
                    buf[p, (e - 2) // 8, (e - 2) % 8,
                        pl.ds(nl * j, nl)] = vs0[j]
                vs0, vs1 = vs1, vs2
            stores(EMB_DIM - 2, vs0)
            stores(EMB_DIM - 1, vs1)

            pltpu.async_copy(
                buf.at[p], out_hbm.at[h, pl.ds(0, ETILES), bt], sem_o)
            return carry

        lax.fori_loop(0, bpw, block_body, 0)

        for _ in range(4):
            pltpu.make_async_copy(
                buf.at[0], out_hbm.at[0, pl.ds(0, ETILES), 0],
                sem_o).wait()

    return emb_kernel, nw


_emb_kernel, _NW = _make_kernel()


def kernel(indices, table):
    idxt = indices.T.reshape(_NW, NBLOCKS // _NW, BBLK)
    out5 = _emb_kernel(idxt, table.T.reshape(EMB_DIM * VOCAB))
    return out5.transpose(2, 4, 0, 1, 3).reshape(BATCH, HIST, EMB_DIM)
